# baseline (speedup 1.0000x reference)
import functools

import jax
import jax.numpy as jnp
from jax import lax
from jax.experimental import pallas as pl
from jax.experimental.pallas import tpu as pltpu
from jax.experimental.pallas import tpu_sc as plsc

_D_MODEL = 64
_BATCH = 16384
_LANES = 16
_CHUNK = 32


def _embed_lookup_t(val, table3):
    info = plsc.get_sparse_core_info()
    num_cores, num_subcores = info.num_cores, info.num_subcores
    num_workers = num_cores * num_subcores
    b_per_w = _BATCH // num_workers
    n_chunks = b_per_w // _CHUNK

    mesh = plsc.VectorSubcoreMesh(core_axis_name="c", subcore_axis_name="s")

    @functools.partial(
        pl.kernel,
        mesh=mesh,
        compiler_params=pltpu.CompilerParams(needs_layout_passes=False),
        out_type=jax.ShapeDtypeStruct((_D_MODEL, _BATCH), jnp.float32),
        scratch_types=[
            pltpu.VMEM((b_per_w,), jnp.int32),
            pltpu.VMEM((b_per_w,), jnp.int32),
            pltpu.VMEM((_CHUNK, 8, _D_MODEL), jnp.float32),
            pltpu.VMEM((_CHUNK, 8, _D_MODEL), jnp.float32),
            pltpu.VMEM((_D_MODEL, b_per_w), jnp.float32),
            pltpu.SemaphoreType.DMA,
            pltpu.SemaphoreType.DMA,
        ],
    )
    def k(idx_hbm, table_hbm, out_hbm, idx_v, tblk_v, blks0, blks1, out_v,
          sem0, sem1):
        wid = lax.axis_index("s") * num_cores + lax.axis_index("c")
        base = wid * b_per_w
        pltpu.sync_copy(idx_hbm.at[pl.ds(base, b_per_w)], idx_v)

        def to_blocks(i, _):
            sl = pl.ds(i * _LANES, _LANES)
            tblk_v[sl] = jax.lax.shift_right_logical(idx_v[sl], 3)
            return _

        lax.fori_loop(0, b_per_w // _LANES, to_blocks, 0, unroll=4)

        lane_iota = lax.iota(jnp.int32, _LANES)

        table_bf = table_hbm.bitcast(jnp.bfloat16)

        def fire(g, blks, sem):
            blks_bf = blks.bitcast(jnp.bfloat16)
            for jg in range(_CHUNK // _LANES):
                t16 = tblk_v[pl.ds(g * _CHUNK + jg * _LANES, _LANES)]
                for lane in range(_LANES):
                    t = jnp.sum(jnp.where(lane_iota == lane, t16, 0))
                    pltpu.async_copy(
                        table_bf.at[t], blks_bf.at[jg * _LANES + lane], sem
                    )

        def drain(blks, sem):
            pltpu.make_async_copy(
                table_bf.at[pl.ds(0, _CHUNK)], blks.bitcast(jnp.bfloat16), sem
            ).wait()

        def extract(g, blks):
            for jg in range(_CHUNK // _LANES):
                sl = pl.ds(g * _CHUNK + jg * _LANES, _LANES)
                r_vec = idx_v[sl] & 7
                j_vec = lax.iota(jnp.int32, _LANES) + jg * _LANES
                for d in range(_D_MODEL):
                    d_vec = jnp.full((_LANES,), d, jnp.int32)
                    row16 = plsc.load_gather(blks, [j_vec, r_vec, d_vec])
                    out_v[d, sl] = row16

        fire(0, blks0, sem0)
        fire(1, blks1, sem1)

        def pair(p, _):
            g0 = p * 2
            g1 = g0 + 1
            drain(blks0, sem0)
            extract(g0, blks0)

            @pl.when(g0 + 2 < n_chunks)
            def _fire0():
                fire(g0 + 2, blks0, sem0)

            drain(blks1, sem1)
            extract(g1, blks1)

            @pl.when(g1 + 2 < n_chunks)
            def _fire1():
                fire(g1 + 2, blks1, sem1)

            return _

        lax.fori_loop(0, n_chunks // 2, pair, 0)
        pltpu.sync_copy(out_v, out_hbm.at[:, pl.ds(base, b_per_w)])

    return k(val, table3)


@jax.jit
def kernel(val, table):
    table3 = table.reshape(table.shape[0] // 8, 8, _D_MODEL)
    out_t = _embed_lookup_t(val.astype(jnp.int32), table3)
    return out_t.T


# speedup vs baseline: 1.0038x; 1.0038x over previous
import functools

import jax
import jax.numpy as jnp
from jax import lax
from jax.experimental import pallas as pl
from jax.experimental.pallas import tpu as pltpu
from jax.experimental.pallas import tpu_sc as plsc

_D_MODEL = 64
_BATCH = 16384
_LANES = 16
_CHUNK = 32


def _embed_lookup_t(val, table3):
    info = plsc.get_sparse_core_info()
    num_cores, num_subcores = info.num_cores, info.num_subcores
    num_workers = num_cores * num_subcores
    b_per_w = _BATCH // num_workers
    n_chunks = b_per_w // _CHUNK

    mesh = plsc.VectorSubcoreMesh(core_axis_name="c", subcore_axis_name="s")

    @functools.partial(
        pl.kernel,
        mesh=mesh,
        compiler_params=pltpu.CompilerParams(needs_layout_passes=False),
        out_type=jax.ShapeDtypeStruct((_D_MODEL, _BATCH), jnp.float32),
        scratch_types=[
            pltpu.VMEM((b_per_w,), jnp.int32),
            pltpu.VMEM((b_per_w,), jnp.int32),
            pltpu.VMEM((_CHUNK, 8, _D_MODEL), jnp.float32),
            pltpu.VMEM((_CHUNK, 8, _D_MODEL), jnp.float32),
            pltpu.VMEM((_D_MODEL, b_per_w), jnp.float32),
            pltpu.SemaphoreType.DMA,
            pltpu.SemaphoreType.DMA,
        ],
    )
    def k(idx_hbm, table_hbm, out_hbm, idx_v, tblk_v, blks0, blks1, out_v,
          sem0, sem1):
        wid = lax.axis_index("s") * num_cores + lax.axis_index("c")
        base = wid * b_per_w
        pltpu.sync_copy(idx_hbm.at[pl.ds(base, b_per_w)], idx_v)

        def to_blocks(i, _):
            sl = pl.ds(i * _LANES, _LANES)
            tblk_v[sl] = jax.lax.shift_right_logical(idx_v[sl], 3)
            return _

        lax.fori_loop(0, b_per_w // _LANES, to_blocks, 0, unroll=4)

        lane_iota = lax.iota(jnp.int32, _LANES)

        def fire(g, blks, sem):
            for jg in range(_CHUNK // _LANES):
                t16 = tblk_v[pl.ds(g * _CHUNK + jg * _LANES, _LANES)]
                ts = [
                    jnp.sum(jnp.where(lane_iota == lane, t16, 0))
                    for lane in range(_LANES)
                ]
                for lane, t in enumerate(ts):
                    pltpu.async_copy(
                        table_hbm.at[t], blks.at[jg * _LANES + lane], sem
                    )

        def drain(blks, sem):
            pltpu.make_async_copy(
                table_hbm.at[pl.ds(0, _CHUNK)], blks, sem
            ).wait()

        def extract(g, blks):
            for jg in range(_CHUNK // _LANES):
                sl = pl.ds(g * _CHUNK + jg * _LANES, _LANES)
                r_vec = idx_v[sl] & 7
                j_vec = lax.iota(jnp.int32, _LANES) + jg * _LANES
                for d in range(_D_MODEL):
                    d_vec = jnp.full((_LANES,), d, jnp.int32)
                    row16 = plsc.load_gather(blks, [j_vec, r_vec, d_vec])
                    out_v[d, sl] = row16

        fire(0, blks0, sem0)
        fire(1, blks1, sem1)

        def pair(p, _):
            g0 = p * 2
            g1 = g0 + 1
            drain(blks0, sem0)
            extract(g0, blks0)

            @pl.when(g0 + 2 < n_chunks)
            def _fire0():
                fire(g0 + 2, blks0, sem0)

            drain(blks1, sem1)
            extract(g1, blks1)

            @pl.when(g1 + 2 < n_chunks)
            def _fire1():
                fire(g1 + 2, blks1, sem1)

            return _

        lax.fori_loop(0, n_chunks // 2, pair, 0)
        pltpu.sync_copy(out_v, out_hbm.at[:, pl.ds(base, b_per_w)])

    return k(val, table3)


@jax.jit
def kernel(val, table):
    table3 = table.reshape(table.shape[0] // 8, 8, _D_MODEL)
    out_t = _embed_lookup_t(val.astype(jnp.int32), table3)
    return out_t.T
